# baseline (speedup 1.0000x reference)
import jax
import jax.numpy as jnp
from jax.experimental import pallas as pl

G = 5


def _fused_kernel(inc_ref, energy_ref, istrack_ref, trackpt_ref, eta_ref,
                  phi_ref, ismuon_ref, layer_ref, feat_ref,
                  w1pa_ref, w1pb_ref, b1p_ref, w2p_ref, b2p_ref, w3p_ref, b3p_ref,
                  w1ca_ref, w1cb_ref, b1c_ref, w2c_ref, b2c_ref, w3c_ref, b3c_ref,
                  outp_ref, outc_ref, topo_ref):
    P = outp_ref.shape[1]
    energy = energy_ref[0]
    isTrack = istrack_ref[0]
    track_pt = trackpt_ref[0]
    eta = eta_ref[0]
    phi = phi_ref[0]
    isMuon = ismuon_ref[0]
    layer = layer_ref[0]

    nt = (isTrack != 1.0).astype(jnp.float32)
    ne = jnp.exp(energy + 1.0) * nt + isTrack * 1e-8
    ct = jnp.concatenate(
        [
            isTrack * track_pt,
            isTrack * eta,
            isTrack * phi,
            isTrack * isMuon,
            ne * (eta * 1.5),
            ne * (phi * 1.8),
            jnp.exp(energy + 2.0) * nt,
            ne * layer,
            ne,
        ],
        axis=0,
    )

    p_all = jnp.dot(ct, inc_ref[0], preferred_element_type=jnp.float32)
    p3 = p_all.reshape(9, G, p_all.shape[1])
    s = p3[:, 0, 0:P]
    for j in range(1, G):
        s = s + p3[:, j, j * P:(j + 1) * P]

    denom = s[8:9]
    eta_s = s[4:5] / denom
    phi_s = s[5:6] / denom
    layer_s = s[7:8] / denom
    energy_s = s[6:7]
    cosh = 0.5 * (jnp.exp(eta_s) + jnp.exp(-eta_s))
    pt = jnp.log(energy_s / cosh) - 2.0
    out8 = jnp.concatenate(
        [s[0:4], pt, eta_s / 1.5, phi_s / 1.8, layer_s], axis=0
    )
    t = out8.T
    skip = t[:, 0:4]
    topo_ref[0] = t[:, 4:8]

    x = feat_ref[0]
    h = jax.nn.relu(x @ w1pa_ref[...] + skip @ w1pb_ref[...] + b1p_ref[...])
    h = jax.nn.relu(h @ w2p_ref[...] + b2p_ref[...])
    outp_ref[0] = h @ w3p_ref[...] + b3p_ref[...]

    h = jax.nn.relu(x @ w1ca_ref[...] + skip @ w1cb_ref[...] + b1c_ref[...])
    h = jax.nn.relu(h @ w2c_ref[...] + b2c_ref[...])
    o = h @ w3c_ref[...] + b3c_ref[...]
    m = jnp.max(o, axis=1, keepdims=True)
    e = jnp.exp(o - m)
    outc_ref[0] = e / jnp.sum(e, axis=1, keepdims=True)


def kernel(features, energy, isTrack, track_pt, eta, phi, isMuon, layer,
           incidence_val, W1p, b1p, W2p, b2p, W3p, b3p, W1c, b1c, W2c, b2c,
           W3c, b3c, edge_src, edge_dst):
    E = incidence_val.shape[0]
    BSN = energy.shape[0]
    BSP, DIM = features.shape
    P = E // BSN
    BS = BSP // P
    N = BSN // BS
    R = N // G
    L = G * P

    inc2 = incidence_val.reshape(BS, R, L)
    node3 = lambda a: a.reshape(BS, R, G).transpose(0, 2, 1)
    nvec = pl.BlockSpec((1, G, R), lambda b: (b, 0, 0))
    const2 = lambda a: pl.BlockSpec(a.shape, lambda b: (0, 0))
    row2 = lambda a: a.reshape(1, -1)

    args = [inc2,
            node3(energy), node3(isTrack), node3(track_pt), node3(eta),
            node3(phi), node3(isMuon), node3(layer),
            features.reshape(BS, P, DIM),
            W1p[:DIM], W1p[DIM:], row2(b1p), W2p, row2(b2p), W3p, row2(b3p),
            W1c[:DIM], W1c[DIM:], row2(b1c), W2c, row2(b2c), W3c, row2(b3c)]
    in_specs = [pl.BlockSpec((1, R, L), lambda b: (b, 0, 0)),
                nvec, nvec, nvec, nvec, nvec, nvec, nvec,
                pl.BlockSpec((1, P, DIM), lambda b: (b, 0, 0))] + \
               [const2(a) for a in args[9:]]

    outp, outc, topo = pl.pallas_call(
        _fused_kernel,
        grid=(BS,),
        in_specs=in_specs,
        out_specs=[
            pl.BlockSpec((1, P, 3), lambda b: (b, 0, 0)),
            pl.BlockSpec((1, P, 6), lambda b: (b, 0, 0)),
            pl.BlockSpec((1, P, 4), lambda b: (b, 0, 0)),
        ],
        out_shape=[
            jax.ShapeDtypeStruct((BS, P, 3), jnp.float32),
            jax.ShapeDtypeStruct((BS, P, 6), jnp.float32),
            jax.ShapeDtypeStruct((BS, P, 4), jnp.float32),
        ],
    )(*args)

    return (outp, outc, topo.reshape(BSP, 4))


# speedup vs baseline: 1.5471x; 1.5471x over previous
import jax
import jax.numpy as jnp
from jax.experimental import pallas as pl

Q = 4


def _agg_kernel(inc0_ref, inc1_ref, inc2_ref, inc3_ref,
                energy_ref, istrack_ref, trackpt_ref, eta_ref,
                phi_ref, ismuon_ref, layer_ref,
                skip_ref, topo_ref):
    energy = energy_ref[0]
    isTrack = istrack_ref[0]
    track_pt = trackpt_ref[0]
    eta = eta_ref[0]
    phi = phi_ref[0]
    isMuon = ismuon_ref[0]
    layer = layer_ref[0]

    nt = (isTrack != 1.0).astype(jnp.float32)
    ne = jnp.exp(energy + 1.0) * nt + isTrack * 1e-8
    ct = jnp.concatenate(
        [
            isTrack * track_pt,
            isTrack * eta,
            isTrack * phi,
            isTrack * isMuon,
            ne * (eta * 1.5),
            ne * (phi * 1.8),
            jnp.exp(energy + 2.0) * nt,
            ne * layer,
            ne,
        ],
        axis=0,
    )

    nq = inc0_ref.shape[2]
    s = jnp.dot(ct[:, 0:nq], inc0_ref[0, 0],
                preferred_element_type=jnp.float32)
    for q, ref in enumerate((inc1_ref, inc2_ref, inc3_ref), start=1):
        s = s + jnp.dot(ct[:, q * nq:(q + 1) * nq], ref[0, 0],
                        preferred_element_type=jnp.float32)

    denom = s[8:9]
    eta_s = s[4:5] / denom
    phi_s = s[5:6] / denom
    layer_s = s[7:8] / denom
    energy_s = s[6:7]
    cosh = 0.5 * (jnp.exp(eta_s) + jnp.exp(-eta_s))
    pt = jnp.log(energy_s / cosh) - 2.0
    out8 = jnp.concatenate(
        [s[0:4], pt, eta_s / 1.5, phi_s / 1.8, layer_s], axis=0
    )
    t = out8.T
    skip_ref[0] = t[:, 0:4]
    topo_ref[0] = t[:, 4:8]


def _heads_kernel(feat_ref, skip_ref,
                  w1pa_ref, w1pb_ref, b1p_ref, w2p_ref, b2p_ref, w3p_ref, b3p_ref,
                  w1ca_ref, w1cb_ref, b1c_ref, w2c_ref, b2c_ref, w3c_ref, b3c_ref,
                  outp_ref, outc_ref):
    x = feat_ref[...]
    skip = skip_ref[...]

    h = jax.nn.relu(x @ w1pa_ref[...] + skip @ w1pb_ref[...] + b1p_ref[...])
    h = jax.nn.relu(h @ w2p_ref[...] + b2p_ref[...])
    outp_ref[...] = h @ w3p_ref[...] + b3p_ref[...]

    h = jax.nn.relu(x @ w1ca_ref[...] + skip @ w1cb_ref[...] + b1c_ref[...])
    h = jax.nn.relu(h @ w2c_ref[...] + b2c_ref[...])
    o = h @ w3c_ref[...] + b3c_ref[...]
    m = jnp.max(o, axis=1, keepdims=True)
    e = jnp.exp(o - m)
    outc_ref[...] = e / jnp.sum(e, axis=1, keepdims=True)


def kernel(features, energy, isTrack, track_pt, eta, phi, isMuon, layer,
           incidence_val, W1p, b1p, W2p, b2p, W3p, b3p, W1c, b1c, W2c, b2c,
           W3c, b3c, edge_src, edge_dst):
    E = incidence_val.shape[0]
    BSN = energy.shape[0]
    BSP, DIM = features.shape
    P = E // BSN
    BS = BSP // P
    N = BSN // BS
    NQ = N // Q

    inc4 = incidence_val.reshape(BS, Q, NQ, P)
    node3 = lambda a: a.reshape(BS, 1, N)
    nvec = pl.BlockSpec((1, 1, N), lambda b: (b, 0, 0))

    def iq(q):
        return pl.BlockSpec((1, 1, NQ, P), lambda b, q=q: (b, q, 0, 0))

    skip, topo = pl.pallas_call(
        _agg_kernel,
        grid=(BS,),
        in_specs=[iq(0), iq(1), iq(2), iq(3),
                  nvec, nvec, nvec, nvec, nvec, nvec, nvec],
        out_specs=[
            pl.BlockSpec((1, P, 4), lambda b: (b, 0, 0)),
            pl.BlockSpec((1, P, 4), lambda b: (b, 0, 0)),
        ],
        out_shape=[
            jax.ShapeDtypeStruct((BS, P, 4), jnp.float32),
            jax.ShapeDtypeStruct((BS, P, 4), jnp.float32),
        ],
    )(inc4, inc4, inc4, inc4,
      node3(energy), node3(isTrack), node3(track_pt), node3(eta),
      node3(phi), node3(isMuon), node3(layer))

    skip2 = skip.reshape(BSP, 4)
    row2 = lambda a: a.reshape(1, -1)
    hargs = [features, skip2,
             W1p[:DIM], W1p[DIM:], row2(b1p), W2p, row2(b2p), W3p, row2(b3p),
             W1c[:DIM], W1c[DIM:], row2(b1c), W2c, row2(b2c), W3c, row2(b3c)]
    outp, outc = pl.pallas_call(
        _heads_kernel,
        in_specs=[pl.BlockSpec(a.shape, lambda: (0,) * a.ndim) for a in hargs],
        out_specs=[
            pl.BlockSpec((BSP, 3), lambda: (0, 0)),
            pl.BlockSpec((BSP, 6), lambda: (0, 0)),
        ],
        out_shape=[
            jax.ShapeDtypeStruct((BSP, 3), jnp.float32),
            jax.ShapeDtypeStruct((BSP, 6), jnp.float32),
        ],
    )(*hargs)

    return (outp.reshape(BS, P, 3), outc.reshape(BS, P, 6),
            topo.reshape(BSP, 4))
